# baseline (speedup 1.0000x reference)
import functools

import jax
import jax.numpy as jnp
from jax.experimental import pallas as pl
from jax.experimental.pallas import tpu as pltpu

_TC = 512


def _moe_body(r_ref, x_ref, router_w_ref, sec_gate_w_ref, up_ref, gate_ref,
              down_ref, out_ref, top_vals_ref, combine_ref, maskall_ref,
              acc_ref, *, E, NSEC, SEC):
    e = pl.program_id(0)
    T, H = x_ref.shape
    FFN = up_ref.shape[1]

    @pl.when(e == 0)
    def _router():
        r = r_ref[...]
        logits = jax.lax.dot_general(
            router_w_ref[...], r, (((1,), (1,)), ((), ())),
            preferred_element_type=jnp.float32)
        sub = jax.lax.broadcasted_iota(jnp.int32, (E, T), 0)
        m1 = jnp.max(logits, axis=0, keepdims=True)
        i1 = jnp.min(jnp.where(logits == m1, sub, E), axis=0, keepdims=True)
        masked = jnp.where(sub == i1, -jnp.inf, logits)
        m2 = jnp.max(masked, axis=0, keepdims=True)
        i2 = jnp.min(jnp.where(masked == m2, sub, E), axis=0, keepdims=True)
        t = jnp.exp(m2 - m1)
        s = 1.0 / (1.0 + t)
        combine_ref[...] = (jnp.where(sub == i1, s, 0.0)
                            + jnp.where(sub == i2, t * s, 0.0))
        top_vals_ref[...] = jnp.transpose(
            jnp.concatenate([m1, m2], axis=0))
        slog = jax.lax.dot_general(
            sec_gate_w_ref[...].reshape(E * NSEC, H), r,
            (((1,), (1,)), ((), ())), preferred_element_type=jnp.float32)
        maskall_ref[...] = (slog > 0.0).astype(jnp.float32)

    up = up_ref[0].astype(jnp.bfloat16)
    gate = gate_ref[0].astype(jnp.bfloat16)
    down = down_ref[0].astype(jnp.bfloat16)

    subj = jax.lax.broadcasted_iota(jnp.int32, (FFN, NSEC), 0)
    lanes = jax.lax.broadcasted_iota(jnp.int32, (FFN, NSEC), 1)
    expand = (subj // SEC == lanes).astype(jnp.bfloat16)

    w_row = combine_ref[pl.ds(e, 1), :]
    mask_e = maskall_ref[pl.ds(e * NSEC, NSEC), :]
    mask_w = (mask_e * w_row).astype(jnp.bfloat16)

    for c in range(T // _TC):
        sl = slice(c * _TC, (c + 1) * _TC)
        x = x_ref[sl, :]
        mask_full = jax.lax.dot_general(
            expand, mask_w[:, sl], (((1,), (0,)), ((), ())),
            preferred_element_type=jnp.float32)
        u = jax.lax.dot_general(up, x, (((1,), (1,)), ((), ())),
                                preferred_element_type=jnp.float32)
        g = jax.lax.dot_general(gate, x, (((1,), (1,)), ((), ())),
                                preferred_element_type=jnp.float32)
        h = (u * mask_full * jnp.maximum(g, 0.0)).astype(jnp.bfloat16)
        y = jax.lax.dot_general(down, h, (((1,), (0,)), ((), ())),
                                preferred_element_type=jnp.float32)

        @pl.when(e == 0)
        def _init():
            acc_ref[:, sl] = y

        @pl.when(e != 0)
        def _acc():
            acc_ref[:, sl] = acc_ref[:, sl] + y

    @pl.when(e == E - 1)
    def _flush():
        out_ref[...] = jnp.transpose(acc_ref[...])


def kernel(router_input, hidden_states, router_w, sec_gate_w, up_w, gate_w, down_w):
    B, S, H = hidden_states.shape
    T = B * S
    E, NSEC, _ = sec_gate_w.shape
    FFN = up_w.shape[1]
    SEC = FFN // NSEC
    x = hidden_states.reshape(T, H)
    r = router_input.reshape(T, H)

    out, top_vals = pl.pallas_call(
        functools.partial(_moe_body, E=E, NSEC=NSEC, SEC=SEC),
        grid=(E,),
        in_specs=[
            pl.BlockSpec((T, H), lambda e: (0, 0)),
            pl.BlockSpec((T, H), lambda e: (0, 0)),
            pl.BlockSpec((E, H), lambda e: (0, 0)),
            pl.BlockSpec((E, NSEC, H), lambda e: (0, 0, 0)),
            pl.BlockSpec((1, FFN, H), lambda e: (e, 0, 0)),
            pl.BlockSpec((1, FFN, H), lambda e: (e, 0, 0)),
            pl.BlockSpec((1, H, FFN), lambda e: (e, 0, 0)),
        ],
        out_specs=[
            pl.BlockSpec((T, H), lambda e: (0, 0)),
            pl.BlockSpec((T, 2), lambda e: (0, 0)),
        ],
        out_shape=[
            jax.ShapeDtypeStruct((T, H), jnp.float32),
            jax.ShapeDtypeStruct((T, 2), jnp.float32),
        ],
        scratch_shapes=[
            pltpu.VMEM((E, T), jnp.float32),
            pltpu.VMEM((E * NSEC, T), jnp.float32),
            pltpu.VMEM((H, T), jnp.float32),
        ],
    )(r, x, router_w, sec_gate_w, up_w, gate_w, down_w)

    return out.reshape(B, S, H), top_vals


# speedup vs baseline: 1.0747x; 1.0747x over previous
import functools

import jax
import jax.numpy as jnp
from jax.experimental import pallas as pl
from jax.experimental.pallas import tpu as pltpu

_TC = 2048


def _moe_body(r_ref, x_ref, router_w_ref, sec_gate_w_ref, up_ref, gate_ref,
              down_ref, out_ref, top_vals_ref, combine_ref, maskall_ref,
              acc_ref, *, E, NSEC, SEC):
    e = pl.program_id(0)
    T, H = x_ref.shape
    FFN = up_ref.shape[1]

    @pl.when(e == 0)
    def _router():
        r = r_ref[...]
        logits = jax.lax.dot_general(
            router_w_ref[...], r, (((1,), (1,)), ((), ())),
            preferred_element_type=jnp.float32)
        sub = jax.lax.broadcasted_iota(jnp.int32, (E, T), 0)
        m1 = jnp.max(logits, axis=0, keepdims=True)
        i1 = jnp.min(jnp.where(logits == m1, sub, E), axis=0, keepdims=True)
        masked = jnp.where(sub == i1, -jnp.inf, logits)
        m2 = jnp.max(masked, axis=0, keepdims=True)
        i2 = jnp.min(jnp.where(masked == m2, sub, E), axis=0, keepdims=True)
        t = jnp.exp(m2 - m1)
        s = 1.0 / (1.0 + t)
        combine_ref[...] = (jnp.where(sub == i1, s, 0.0)
                            + jnp.where(sub == i2, t * s, 0.0))
        top_vals_ref[...] = jnp.transpose(
            jnp.concatenate([m1, m2], axis=0))
        slog = jax.lax.dot_general(
            sec_gate_w_ref[...].reshape(E * NSEC, H), r,
            (((1,), (1,)), ((), ())), preferred_element_type=jnp.float32)
        maskall_ref[...] = (slog > 0.0).astype(jnp.float32)

    up = up_ref[0].astype(jnp.bfloat16)
    gate = gate_ref[0].astype(jnp.bfloat16)
    down = down_ref[0].astype(jnp.bfloat16)

    subj = jax.lax.broadcasted_iota(jnp.int32, (FFN, NSEC), 0)
    lanes = jax.lax.broadcasted_iota(jnp.int32, (FFN, NSEC), 1)
    expand = (subj // SEC == lanes).astype(jnp.bfloat16)

    w_row = combine_ref[pl.ds(e, 1), :]
    mask_e = maskall_ref[pl.ds(e * NSEC, NSEC), :]
    mask_w = (mask_e * w_row).astype(jnp.bfloat16)

    for c in range(T // _TC):
        sl = slice(c * _TC, (c + 1) * _TC)
        x = x_ref[sl, :]
        mask_full = jax.lax.dot_general(
            expand, mask_w[:, sl], (((1,), (0,)), ((), ())),
            preferred_element_type=jnp.float32)
        u = jax.lax.dot_general(up, x, (((1,), (1,)), ((), ())),
                                preferred_element_type=jnp.float32)
        g = jax.lax.dot_general(gate, x, (((1,), (1,)), ((), ())),
                                preferred_element_type=jnp.float32)
        h = (u * mask_full * jnp.maximum(g, 0.0)).astype(jnp.bfloat16)
        y = jax.lax.dot_general(down, h, (((1,), (0,)), ((), ())),
                                preferred_element_type=jnp.float32)

        @pl.when(e == 0)
        def _init():
            acc_ref[:, sl] = y

        @pl.when(e != 0)
        def _acc():
            acc_ref[:, sl] = acc_ref[:, sl] + y

    @pl.when(e == E - 1)
    def _flush():
        out_ref[...] = jnp.transpose(acc_ref[...])


def kernel(router_input, hidden_states, router_w, sec_gate_w, up_w, gate_w, down_w):
    B, S, H = hidden_states.shape
    T = B * S
    E, NSEC, _ = sec_gate_w.shape
    FFN = up_w.shape[1]
    SEC = FFN // NSEC
    x = hidden_states.reshape(T, H)
    r = router_input.reshape(T, H)

    out, top_vals = pl.pallas_call(
        functools.partial(_moe_body, E=E, NSEC=NSEC, SEC=SEC),
        grid=(E,),
        in_specs=[
            pl.BlockSpec((T, H), lambda e: (0, 0)),
            pl.BlockSpec((T, H), lambda e: (0, 0)),
            pl.BlockSpec((E, H), lambda e: (0, 0)),
            pl.BlockSpec((E, NSEC, H), lambda e: (0, 0, 0)),
            pl.BlockSpec((1, FFN, H), lambda e: (e, 0, 0)),
            pl.BlockSpec((1, FFN, H), lambda e: (e, 0, 0)),
            pl.BlockSpec((1, H, FFN), lambda e: (e, 0, 0)),
        ],
        out_specs=[
            pl.BlockSpec((T, H), lambda e: (0, 0)),
            pl.BlockSpec((T, 2), lambda e: (0, 0)),
        ],
        out_shape=[
            jax.ShapeDtypeStruct((T, H), jnp.float32),
            jax.ShapeDtypeStruct((T, 2), jnp.float32),
        ],
        scratch_shapes=[
            pltpu.VMEM((E, T), jnp.float32),
            pltpu.VMEM((E * NSEC, T), jnp.float32),
            pltpu.VMEM((H, T), jnp.float32),
        ],
    )(r, x, router_w, sec_gate_w, up_w, gate_w, down_w)

    return out.reshape(B, S, H), top_vals
